# baseline (speedup 1.0000x reference)
import jax
import jax.numpy as jnp
from jax import lax
from jax.experimental import pallas as pl
from jax.experimental.pallas import tpu as pltpu
from jax.experimental.pallas import tpu_sc as plsc


def _sc_gather(t, taba, tabb):
    batch = t.shape[0]
    info = plsc.get_sparse_core_info()
    nw = info.num_cores * info.num_subcores
    bw = batch // nw
    mesh = plsc.VectorSubcoreMesh(core_axis_name="c", subcore_axis_name="s")

    def body(t_hbm, taba_hbm, tabb_hbm, a_hbm, b_hbm, idx_v, av, bv, sema, semb):
        wid = lax.axis_index("s") * info.num_cores + lax.axis_index("c")
        base = wid * bw
        pltpu.sync_copy(t_hbm.at[pl.ds(base, bw)], idx_v)
        ca = pltpu.async_copy(taba_hbm.at[idx_v], av, sema)
        cb = pltpu.async_copy(tabb_hbm.at[idx_v], bv, semb)
        ca.wait()
        cb.wait()
        pltpu.sync_copy(av, a_hbm.at[pl.ds(base, bw)])
        pltpu.sync_copy(bv, b_hbm.at[pl.ds(base, bw)])

    return pl.kernel(
        body,
        out_type=(
            jax.ShapeDtypeStruct((batch,), jnp.float32),
            jax.ShapeDtypeStruct((batch,), jnp.float32),
        ),
        mesh=mesh,
        scratch_types=[
            pltpu.VMEM((bw,), jnp.int32),
            pltpu.VMEM((bw,), jnp.float32),
            pltpu.VMEM((bw,), jnp.float32),
            pltpu.SemaphoreType.DMA,
            pltpu.SemaphoreType.DMA,
        ],
    )(t, taba, tabb)


def _fma_body(a_ref, b_ref, x_ref, n_ref, o_ref):
    a = a_ref[...][:, None]
    b = b_ref[...][:, None]
    o_ref[...] = a * x_ref[...] + b * n_ref[...]


def kernel(x_start, t, noise, sqrt_alphas_cumprod, sqrt_one_minus_alphas_cumprod):
    batch, dim = x_start.shape
    a, b = _sc_gather(t.astype(jnp.int32), sqrt_alphas_cumprod,
                      sqrt_one_minus_alphas_cumprod)
    br = 2048
    nb = batch // br
    return pl.pallas_call(
        _fma_body,
        grid=(nb,),
        in_specs=[
            pl.BlockSpec((br,), lambda i: (i,)),
            pl.BlockSpec((br,), lambda i: (i,)),
            pl.BlockSpec((br, dim), lambda i: (i, 0)),
            pl.BlockSpec((br, dim), lambda i: (i, 0)),
        ],
        out_specs=pl.BlockSpec((br, dim), lambda i: (i, 0)),
        out_shape=jax.ShapeDtypeStruct((batch, dim), jnp.float32),
    )(a, b, x_start, noise)


# speedup vs baseline: 1.3631x; 1.3631x over previous
import jax
import jax.numpy as jnp
from jax.experimental import pallas as pl

_NUM_SCALES = 1000
_TPAD = 1024


def _body(t_ref, tab_ref, x_ref, n_ref, o_ref):
    tcol = t_ref[...][:, None]
    br = tcol.shape[0]
    hi = tcol >> 7
    lo = tcol & 127
    oh_hi = (hi == jax.lax.broadcasted_iota(jnp.int32, (br, 8), 1)).astype(jnp.float32)
    rows = jnp.dot(oh_hi, tab_ref[...], preferred_element_type=jnp.float32,
                   precision=jax.lax.Precision.HIGHEST)
    a = jnp.take_along_axis(rows[:, :128], lo, axis=1)
    b = jnp.take_along_axis(rows[:, 128:], lo, axis=1)
    o_ref[...] = a * x_ref[...] + b * n_ref[...]


def kernel(x_start, t, noise, sqrt_alphas_cumprod, sqrt_one_minus_alphas_cumprod):
    batch, dim = x_start.shape
    br = 2048
    nb = batch // br
    t1 = t.astype(jnp.int32)
    taba = jnp.zeros((_TPAD,), jnp.float32).at[:_NUM_SCALES].set(
        sqrt_alphas_cumprod).reshape(8, 128)
    tabb = jnp.zeros((_TPAD,), jnp.float32).at[:_NUM_SCALES].set(
        sqrt_one_minus_alphas_cumprod).reshape(8, 128)
    tab = jnp.concatenate([taba, tabb], axis=1)
    return pl.pallas_call(
        _body,
        grid=(nb,),
        in_specs=[
            pl.BlockSpec((br,), lambda i: (i,)),
            pl.BlockSpec((8, 256), lambda i: (0, 0)),
            pl.BlockSpec((br, dim), lambda i: (i, 0)),
            pl.BlockSpec((br, dim), lambda i: (i, 0)),
        ],
        out_specs=pl.BlockSpec((br, dim), lambda i: (i, 0)),
        out_shape=jax.ShapeDtypeStruct((batch, dim), jnp.float32),
    )(t1, tab, x_start, noise)


# speedup vs baseline: 2.7066x; 1.9856x over previous
import jax
import jax.numpy as jnp
from jax.experimental import pallas as pl

_NUM_SCALES = 1000
_TPAD = 1024


def _body(t_ref, tab_ref, x_ref, n_ref, o_ref):
    tcol = t_ref[...][:, None]
    br = tcol.shape[0]
    hi = tcol >> 7
    lo = tcol & 127
    oh_hi = (hi == jax.lax.broadcasted_iota(jnp.int32, (br, 8), 1)).astype(jnp.float32)
    oh_lo = lo == jax.lax.broadcasted_iota(jnp.int32, (br, 128), 1)
    rows = jnp.dot(oh_hi, tab_ref[...], preferred_element_type=jnp.float32)
    a = jnp.sum(jnp.where(oh_lo, rows[:, :128], 0.0), axis=1, keepdims=True)
    b = jnp.sum(jnp.where(oh_lo, rows[:, 128:], 0.0), axis=1, keepdims=True)
    o_ref[...] = a * x_ref[...] + b * n_ref[...]


def kernel(x_start, t, noise, sqrt_alphas_cumprod, sqrt_one_minus_alphas_cumprod):
    batch, dim = x_start.shape
    br = 2048
    nb = batch // br
    t1 = t.astype(jnp.int32)
    taba = jnp.zeros((_TPAD,), jnp.float32).at[:_NUM_SCALES].set(
        sqrt_alphas_cumprod).reshape(8, 128)
    tabb = jnp.zeros((_TPAD,), jnp.float32).at[:_NUM_SCALES].set(
        sqrt_one_minus_alphas_cumprod).reshape(8, 128)
    tab = jnp.concatenate([taba, tabb], axis=1)
    return pl.pallas_call(
        _body,
        grid=(nb,),
        in_specs=[
            pl.BlockSpec((br,), lambda i: (i,)),
            pl.BlockSpec((8, 256), lambda i: (0, 0)),
            pl.BlockSpec((br, dim), lambda i: (i, 0)),
            pl.BlockSpec((br, dim), lambda i: (i, 0)),
        ],
        out_specs=pl.BlockSpec((br, dim), lambda i: (i, 0)),
        out_shape=jax.ShapeDtypeStruct((batch, dim), jnp.float32),
    )(t1, tab, x_start, noise)


# speedup vs baseline: 2.9196x; 1.0787x over previous
import jax
import jax.numpy as jnp
from jax.experimental import pallas as pl

_NUM_SCALES = 1000
_TPAD = 1024


def _body(t_ref, tab_ref, x_ref, n_ref, o_ref):
    tcol = t_ref[...][:, None]
    br = tcol.shape[0]
    hi = tcol >> 7
    lo = tcol & 127
    oh_hi = (hi == jax.lax.broadcasted_iota(jnp.int32, (br, 8), 1)).astype(jnp.float32)
    oh_lo = lo == jax.lax.broadcasted_iota(jnp.int32, (br, 128), 1)
    rows = jnp.dot(oh_hi, tab_ref[...], preferred_element_type=jnp.float32)
    a = jnp.sum(jnp.where(oh_lo, rows[:, :128], 0.0), axis=1, keepdims=True)
    b = jnp.sum(jnp.where(oh_lo, rows[:, 128:], 0.0), axis=1, keepdims=True)
    o_ref[...] = a * x_ref[...] + b * n_ref[...]


def kernel(x_start, t, noise, sqrt_alphas_cumprod, sqrt_one_minus_alphas_cumprod):
    batch, dim = x_start.shape
    br = 4096
    nb = batch // br
    t1 = t.astype(jnp.int32)
    taba = jnp.zeros((_TPAD,), jnp.float32).at[:_NUM_SCALES].set(
        sqrt_alphas_cumprod).reshape(8, 128)
    tabb = jnp.zeros((_TPAD,), jnp.float32).at[:_NUM_SCALES].set(
        sqrt_one_minus_alphas_cumprod).reshape(8, 128)
    tab = jnp.concatenate([taba, tabb], axis=1)
    return pl.pallas_call(
        _body,
        grid=(nb,),
        in_specs=[
            pl.BlockSpec((br,), lambda i: (i,)),
            pl.BlockSpec((8, 256), lambda i: (0, 0)),
            pl.BlockSpec((br, dim), lambda i: (i, 0)),
            pl.BlockSpec((br, dim), lambda i: (i, 0)),
        ],
        out_specs=pl.BlockSpec((br, dim), lambda i: (i, 0)),
        out_shape=jax.ShapeDtypeStruct((batch, dim), jnp.float32),
    )(t1, tab, x_start, noise)
